# baseline (speedup 1.0000x reference)
import functools

import jax
import jax.numpy as jnp
from jax import lax
from jax.experimental import pallas as pl
from jax.experimental.pallas import tpu as pltpu
from jax.experimental.pallas import tpu_sc as plsc

_EMB_D = 128
_LANES = 16
_GROUPS = _EMB_D // _LANES
_NBUF = 6
_CHUNK = 64


@functools.lru_cache(maxsize=None)
def _make_gather(bs: int, seq: int):
    info = plsc.get_sparse_core_info()
    nc, ns = info.num_cores, info.num_subcores
    nw = nc * ns
    total = bs * seq
    n_chunks_total = total // _CHUNK
    n_chunks = n_chunks_total // nw
    chunks_per_s = bs // _CHUNK
    assert total % (_CHUNK * nw) == 0 and bs % _CHUNK == 0
    seq_pad = (seq + 7) // 8 * 8

    mesh = plsc.VectorSubcoreMesh(core_axis_name="c", subcore_axis_name="s")

    @functools.partial(
        pl.kernel,
        out_type=jax.ShapeDtypeStruct((total, _EMB_D), jnp.float32),
        mesh=mesh,
        cost_estimate=pl.CostEstimate(
            flops=2 * total * _EMB_D,
            bytes_accessed=2 * total * _EMB_D * 4,
            transcendentals=0,
        ),
        scratch_types=[
            pltpu.VMEM((n_chunks, _CHUNK), jnp.int32),
            pltpu.VMEM((seq_pad, _EMB_D), jnp.float32),
            pltpu.VMEM((_NBUF, _CHUNK, _EMB_D), jnp.float32),
            pltpu.SemaphoreType.DMA((_NBUF,)),
            pltpu.SemaphoreType.DMA((_NBUF,)),
        ],
    )
    def gather_kernel(table_hbm, idx_hbm, pos_hbm, out_hbm,
                      idx_v, pos_v, buf, gsem, wsem):
        wid = lax.axis_index("s") * nc + lax.axis_index("c")
        c0 = wid * n_chunks
        pltpu.sync_copy(idx_hbm.at[wid], idx_v)
        pltpu.sync_copy(pos_hbm.at[pl.ds(0, seq_pad)], pos_v)

        def issue_gather(j):
            p = j % _NBUF
            return pltpu.async_copy(table_hbm.at[idx_v.at[j]], buf.at[p],
                                    gsem.at[p])

        gdesc = [None] * n_chunks
        wdesc = [None] * n_chunks
        gdesc[0] = issue_gather(0)
        gdesc[1] = issue_gather(1)
        gdesc[2] = issue_gather(2)
        for j in range(n_chunks):
            p = j % _NBUF
            gdesc[j].wait()
            s = (c0 + j) // chunks_per_s
            prow = [pos_v[s, pl.ds(g * _LANES, _LANES)] for g in range(_GROUPS)]

            @pl.loop(0, _CHUNK)
            def _(r):
                for g in range(_GROUPS):
                    plsc.addupdate(buf.at[p, r, pl.ds(g * _LANES, _LANES)],
                                   prow[g])

            if j + 3 < n_chunks:
                if j >= 3:
                    wdesc[j - 3].wait()
                gdesc[j + 3] = issue_gather(j + 3)
            wdesc[j] = pltpu.async_copy(
                buf.at[p],
                out_hbm.at[pl.ds((c0 + j) * _CHUNK, _CHUNK)],
                wsem.at[p])
        for j in range(n_chunks - 3, n_chunks):
            wdesc[j].wait()

    def run(table, ids, pos):
        ids_smajor = ids.T.reshape(nw, n_chunks, _CHUNK)
        out = gather_kernel(table, ids_smajor, pos)
        return out.reshape(seq, bs, _EMB_D).transpose(1, 0, 2)

    return run


def kernel(inputs, embedding_table, pos_emb_cache):
    bs, seq = inputs.shape
    vocab, d = embedding_table.shape
    gather = _make_gather(bs, seq)
    rv = (inputs[0, :] >= 0).astype(jnp.float32)
    zv = (inputs[:, 0] < 0).astype(jnp.float32)
    pos_emb = (jnp.broadcast_to(pos_emb_cache[None, :seq, :], (bs, seq, d))
               + jnp.broadcast_to(zv[:, None, None], (bs, seq, d)))
    mask = (inputs > 0).astype(jnp.int32)
    attn_pattern_mask = jnp.maximum(
        jnp.broadcast_to(rv[None, None, :, None], (bs, 4, seq, seq)),
        jnp.broadcast_to(rv[None, None, None, :], (bs, 4, seq, seq)))
    x = gather(embedding_table, inputs, pos_emb_cache)
    modality_index = jnp.array(0, dtype=jnp.int32)
    return (x, pos_emb, modality_index, mask, attn_pattern_mask)
